# Initial kernel scaffold; baseline (speedup 1.0000x reference)
import jax
import jax.numpy as jnp
from jax.experimental import pallas as pl


def kernel(x, Wg, bg, W1, b1, W2, b2):
    raise NotImplementedError("write your pallas kernel here")


# speedup vs baseline: 2.1559x; 2.1559x over previous
import functools

import jax
import jax.numpy as jnp
from jax.experimental import pallas as pl


def _moe_kernel(x_ref, wg_ref, bg_ref, w1_ref, b1_ref, w2_ref, b2_ref, o_ref):
    xb = x_ref[...]
    logits = jnp.dot(xb, wg_ref[...], preferred_element_type=jnp.float32)
    logits = logits + bg_ref[...]
    e = logits.shape[-1]
    v1 = jnp.max(logits, axis=-1, keepdims=True)
    iota = jax.lax.broadcasted_iota(jnp.int32, logits.shape, 1)
    idx1 = jnp.min(jnp.where(logits >= v1, iota, e), axis=-1, keepdims=True)
    v2 = jnp.max(jnp.where(iota == idx1, -jnp.inf, logits), axis=-1, keepdims=True)
    scale = 2.0 - 1.0 / (1.0 + jnp.exp(v2 - v1))
    h1 = jnp.dot(xb, w1_ref[...], preferred_element_type=jnp.float32)
    h1 = jnp.maximum(h1 + b1_ref[...], 0.0)
    h = jnp.dot(h1, w2_ref[...], preferred_element_type=jnp.float32)
    o_ref[...] = scale * (h + b2_ref[...])


@functools.partial(jax.jit, static_argnames=())
def kernel(x, Wg, bg, W1, b1, W2, b2):
    B, N, D = x.shape
    T = B * N
    E = Wg.shape[1]
    H = W1.shape[1]
    BT = 512
    xf = x.reshape(T, D)

    out = pl.pallas_call(
        _moe_kernel,
        grid=(T // BT,),
        in_specs=[
            pl.BlockSpec((BT, D), lambda i: (i, 0)),
            pl.BlockSpec((D, E), lambda i: (0, 0)),
            pl.BlockSpec((1, E), lambda i: (0, 0)),
            pl.BlockSpec((D, H), lambda i: (0, 0)),
            pl.BlockSpec((1, H), lambda i: (0, 0)),
            pl.BlockSpec((H, D), lambda i: (0, 0)),
            pl.BlockSpec((1, D), lambda i: (0, 0)),
        ],
        out_specs=pl.BlockSpec((BT, D), lambda i: (i, 0)),
        out_shape=jax.ShapeDtypeStruct((T, D), x.dtype),
    )(xf, Wg, bg.reshape(1, E), W1, b1.reshape(1, H), W2, b2.reshape(1, D))
    return out.reshape(B, N, D)


# speedup vs baseline: 2.1651x; 1.0043x over previous
import functools

import jax
import jax.numpy as jnp
from jax.experimental import pallas as pl


def _moe_kernel(x_ref, wg_ref, bg_ref, w1_ref, b1_ref, w2_ref, b2_ref, o_ref):
    xb = x_ref[...]
    logits = jnp.dot(xb, wg_ref[...], preferred_element_type=jnp.float32)
    logits = logits + bg_ref[...]
    e = logits.shape[-1]
    v1 = jnp.max(logits, axis=-1, keepdims=True)
    iota = jax.lax.broadcasted_iota(jnp.int32, logits.shape, 1)
    idx1 = jnp.min(jnp.where(logits >= v1, iota, e), axis=-1, keepdims=True)
    v2 = jnp.max(jnp.where(iota == idx1, -jnp.inf, logits), axis=-1, keepdims=True)
    scale = 2.0 - 1.0 / (1.0 + jnp.exp(v2 - v1))
    h1 = jnp.dot(xb.astype(jnp.bfloat16), w1_ref[...].astype(jnp.bfloat16),
                 preferred_element_type=jnp.float32)
    h1 = jnp.maximum(h1 + b1_ref[...], 0.0)
    h = jnp.dot(h1.astype(jnp.bfloat16), w2_ref[...].astype(jnp.bfloat16),
                preferred_element_type=jnp.float32)
    o_ref[...] = scale * (h + b2_ref[...])


@functools.partial(jax.jit, static_argnames=())
def kernel(x, Wg, bg, W1, b1, W2, b2):
    B, N, D = x.shape
    T = B * N
    E = Wg.shape[1]
    H = W1.shape[1]
    BT = 512
    xf = x.reshape(T, D)

    out = pl.pallas_call(
        _moe_kernel,
        grid=(T // BT,),
        in_specs=[
            pl.BlockSpec((BT, D), lambda i: (i, 0)),
            pl.BlockSpec((D, E), lambda i: (0, 0)),
            pl.BlockSpec((1, E), lambda i: (0, 0)),
            pl.BlockSpec((D, H), lambda i: (0, 0)),
            pl.BlockSpec((1, H), lambda i: (0, 0)),
            pl.BlockSpec((H, D), lambda i: (0, 0)),
            pl.BlockSpec((1, D), lambda i: (0, 0)),
        ],
        out_specs=pl.BlockSpec((BT, D), lambda i: (i, 0)),
        out_shape=jax.ShapeDtypeStruct((T, D), x.dtype),
    )(xf, Wg, bg.reshape(1, E), W1, b1.reshape(1, H), W2, b2.reshape(1, D))
    return out.reshape(B, N, D)
